# baseline (speedup 1.0000x reference)
import functools

import jax
import jax.numpy as jnp
from jax import lax
from jax.experimental import pallas as pl
from jax.experimental.pallas import tpu as pltpu
from jax.experimental.pallas import tpu_sc as plsc

N_ITEMS = 1000000
N_FACTORS = 64
BATCH = 16384

_info = plsc.get_sparse_core_info()
NC, NS, L = _info.num_cores, _info.num_subcores, _info.num_lanes
NW = NC * NS

NJ = 7813
CPT = 248
NCOLS = 4
W = NCOLS * 128
NCHUNK = CPT // NCOLS
JC_MAX = NJ - NCOLS
NBUF = 3

NGRP = 8
GCH = 8
GCOLS = NCOLS * GCH

PAIR_CAP = 1536
GRP_CAP = 256
ACT_CAP = 128
NPR = BATCH * 2 + 8
DUMP = BATCH * 2

ICH = 2048


def _extract_body(items0_hbm, items1_hbm, table_hbm, rows_hbm,
                  sbuf, ibuf, pairs_r, pairs_pay, grp_r, grp_pay,
                  actv_r, actv_pay, stag, payidx, sem_in, sem_sc, sem_i):
    cid = lax.axis_index("c")
    sid = lax.axis_index("s")
    wid = sid * NC + cid
    jlo = wid * CPT

    lane = lax.iota(jnp.int32, L)

    def compact_chunk(np_cur, items_hbm, slot, cb, ib, nxt):
        pltpu.make_async_copy(
            items_hbm.at[pl.ds(0, ICH)], ibuf.at[ib], sem_i).wait()
        if nxt is not None:
            pltpu.async_copy(
                nxt[0].at[pl.ds(nxt[1], ICH)], ibuf.at[1 - ib], sem_i)

        def vbody(v, np_c):
            r16 = ibuf[ib, pl.ds(pl.multiple_of(v * L, L), L)]
            j16 = lax.shift_right_logical(r16, 7)
            mask = (j16 >= jlo) & (j16 < jlo + CPT)
            pay16 = jnp.full((L,), slot * BATCH + cb, jnp.int32) + v * L + lane
            off = pl.multiple_of(0, 1) + np_c
            plsc.store_compressed(pairs_r.at[pl.ds(off, L)], r16, mask=mask)
            plsc.store_compressed(pairs_pay.at[pl.ds(off, L)], pay16,
                                  mask=mask)
            cnt = plsc.all_reduce_population_count(mask)[0]
            return np_c + cnt

        return lax.fori_loop(0, ICH // L, vbody, np_cur, unroll=False)

    np_total = 0
    steps = [(slot, ih, c * ICH)
             for slot, ih in ((0, items0_hbm), (1, items1_hbm))
             for c in range(BATCH // ICH)]
    pltpu.async_copy(items0_hbm.at[pl.ds(0, ICH)], ibuf.at[0], sem_i)
    for k, (slot, ih, cb) in enumerate(steps):
        nxt = (steps[k + 1][1], steps[k + 1][2]) if k + 1 < len(steps) else None
        np_total = compact_chunk(np_total, ih, slot, cb, k % 2, nxt)
    np_total = jnp.minimum(np_total, PAIR_CAP - L)
    nq_all = lax.shift_right_logical(np_total + L - 1, 4)

    gcnts = []
    for grp in range(NGRP):
        glo = jlo + grp * GCOLS

        def g2body(q, ng_c, glo=glo, grp=grp):
            off_q = pl.multiple_of(0, 1) + q * L
            r16 = pairs_r[pl.ds(off_q, L)]
            p16 = pairs_pay[pl.ds(off_q, L)]
            j16 = lax.shift_right_logical(r16, 7)
            valid = (q * L + lane) < np_total
            mask = valid & (j16 >= glo) & (j16 < glo + GCOLS)
            off = pl.multiple_of(0, 1) + ng_c
            plsc.store_compressed(grp_r.at[grp, pl.ds(off, L)], r16,
                                  mask=mask)
            plsc.store_compressed(grp_pay.at[grp, pl.ds(off, L)], p16,
                                  mask=mask)
            return ng_c + plsc.all_reduce_population_count(mask)[0]

        cnt_g = lax.fori_loop(0, nq_all, g2body, 0, unroll=False)
        gcnts.append(jnp.minimum(cnt_g, GRP_CAP - L))

    kvecs = [lane + g * L for g in range(4)]
    i_vecs = [lax.shift_right_logical(k, 3) for k in kvecs]
    d_vecs = [lax.bitwise_and(k, 7) for k in kvecs]

    def fire_chunk(k):
        jc_eff = jnp.minimum(jlo + k * NCOLS, JC_MAX)
        buf = lax.rem(k, NBUF)
        for i in range(8):
            pltpu.async_copy(
                table_hbm.at[pl.ds(i * 8, 8), pl.ds(jc_eff * 128, W)],
                sbuf.at[buf, i], sem_in)

    for k0 in range(NBUF):
        fire_chunk(jnp.int32(k0))

    def drain_scatter():
        pltpu.make_async_copy(
            rows_hbm.at[pl.ds(0, L)], stag.at[0], sem_sc).wait()

    for grp in range(NGRP):
        cnt_g = gcnts[grp]
        nq_g = lax.shift_right_logical(cnt_g + L - 1, 4)

        def chunk_body(cc, _, grp=grp, cnt_g=cnt_g, nq_g=nq_g):
            k = grp * GCH + cc
            jc = jlo + k * NCOLS
            jc_eff = jnp.minimum(jc, JC_MAX)
            buf = lax.rem(k, NBUF)
            for i in range(8):
                pltpu.make_async_copy(
                    table_hbm.at[pl.ds(i * 8, 8), pl.ds(0, W)],
                    sbuf.at[buf, i], sem_in).wait()

            def sel_body(q, na_c):
                off_q = pl.multiple_of(0, 1) + q * L
                r16 = grp_r[grp, pl.ds(off_q, L)]
                p16 = grp_pay[grp, pl.ds(off_q, L)]
                j16 = lax.shift_right_logical(r16, 7)
                valid = (q * L + lane) < cnt_g
                mask = valid & (j16 >= jc) & (j16 < jc + NCOLS)
                off = pl.multiple_of(0, 1) + na_c
                plsc.store_compressed(actv_r.at[pl.ds(off, L)], r16,
                                      mask=mask)
                plsc.store_compressed(actv_pay.at[pl.ds(off, L)], p16,
                                      mask=mask)
                return na_c + plsc.all_reduce_population_count(mask)[0]

            fill_r = jnp.full((L,), 0, jnp.int32) + jc_eff * 128
            fill_p = jnp.full((L,), DUMP, jnp.int32)
            for q in range(ACT_CAP // L):
                actv_r[pl.ds(q * L, L)] = fill_r
                actv_pay[pl.ds(q * L, L)] = fill_p

            na = lax.fori_loop(0, nq_g, sel_body, 0, unroll=False)
            na = jnp.minimum(na, ACT_CAP - L)
            ng = lax.shift_right_logical(na + L - 1, 4)

            def grp_body(g, _g):
                slot_g = lax.rem(g, 4)

                @pl.when(g >= 4)
                def _():
                    drain_scatter()

                goff = pl.multiple_of(0, 1) + g * L
                r16 = actv_r[pl.ds(goff, L)]
                p16 = actv_pay[pl.ds(goff, L)]
                payidx[slot_g] = p16
                for l in range(L):
                    pos = r16[l] - jc_eff * 128
                    posv = jnp.full((L,), 0, jnp.int32) + pos
                    for gg in range(4):
                        vals = plsc.load_gather(
                            sbuf.at[buf], [i_vecs[gg], d_vecs[gg], posv])
                        stag[slot_g, l, pl.ds(gg * L, L)] = vals
                pltpu.async_copy(
                    stag.at[slot_g], rows_hbm.at[payidx.at[slot_g]], sem_sc)
                return 0

            lax.fori_loop(0, ng, grp_body, 0, unroll=False)

            def drain_body(g, _g):
                drain_scatter()
                return 0

            lax.fori_loop(0, jnp.minimum(ng, 4), drain_body, 0,
                          unroll=False)

            @pl.when(k + NBUF < NCHUNK)
            def _():
                fire_chunk(k + NBUF)

            return 0

        nch_grp = min(GCH, NCHUNK - grp * GCH)
        lax.fori_loop(0, nch_grp, chunk_body, 0, unroll=False)


@jax.jit
def _sc_extract(items0, items1, table_t):
    mesh = plsc.VectorSubcoreMesh(core_axis_name="c", subcore_axis_name="s")
    f = functools.partial(
        pl.kernel,
        out_type=jax.ShapeDtypeStruct((NPR, 128), jnp.float32),
        mesh=mesh,
        scratch_types=[
            pltpu.VMEM((NBUF, 8, 8, W), jnp.float32),
            pltpu.VMEM((2, ICH), jnp.int32),
            pltpu.VMEM((PAIR_CAP,), jnp.int32),
            pltpu.VMEM((PAIR_CAP,), jnp.int32),
            pltpu.VMEM((NGRP, GRP_CAP), jnp.int32),
            pltpu.VMEM((NGRP, GRP_CAP), jnp.int32),
            pltpu.VMEM((ACT_CAP,), jnp.int32),
            pltpu.VMEM((ACT_CAP,), jnp.int32),
            pltpu.VMEM((4, L, 128), jnp.float32),
            pltpu.VMEM((4, L), jnp.int32),
            pltpu.SemaphoreType.DMA,
            pltpu.SemaphoreType.DMA,
            pltpu.SemaphoreType.DMA,
        ],
        compiler_params=pltpu.CompilerParams(
            use_tc_tiling_on_sc=True,
            needs_layout_passes=False,
        ),
    )(_extract_body)
    return f(items0, items1, table_t)


def _combine_body(a_ref, b_ref, o_ref):
    x = a_ref[:, :N_FACTORS] * b_ref[:, :N_FACTORS]
    s = jnp.sum(x, axis=1)
    o_ref[...] = 1.0 / (1.0 + jnp.exp(-s))


_BLK = 2048


@jax.jit
def _tc_combine(rows):
    return pl.pallas_call(
        _combine_body,
        grid=(BATCH // _BLK,),
        in_specs=[
            pl.BlockSpec((_BLK, 128), lambda i: (i, 0)),
            pl.BlockSpec((_BLK, 128), lambda i: (i + BATCH // _BLK, 0)),
        ],
        out_specs=pl.BlockSpec((_BLK,), lambda i: (i,)),
        out_shape=jax.ShapeDtypeStruct((BATCH,), jnp.float32),
    )(rows, rows)


def kernel(items, item_factors):
    items0 = items[0].astype(jnp.int32)
    items1 = items[1].astype(jnp.int32)
    table_t = item_factors.T
    rows = _sc_extract(items0, items1, table_t)
    return _tc_combine(rows)


# speedup vs baseline: 2.0421x; 2.0421x over previous
import functools

import jax
import jax.numpy as jnp
from jax import lax
from jax.experimental import pallas as pl
from jax.experimental.pallas import tpu as pltpu
from jax.experimental.pallas import tpu_sc as plsc

N_ITEMS = 1000000
N_FACTORS = 64
TILE_H = 8
N_TILES = N_ITEMS // TILE_H
BATCH = 16384

_info = plsc.get_sparse_core_info()
NC, NS, L = _info.num_cores, _info.num_subcores, _info.num_lanes
NW = NC * NS
BW = BATCH // NW
PH = 32
NPH = BW // PH
NG = PH // L


def _body(items0_hbm, items1_hbm, table_hbm, out_hbm,
          raw0_v, raw1_v, rows0_v, rows1_v, out_v,
          sem0, sem1):
    cid = lax.axis_index("c")
    sid = lax.axis_index("s")
    wid = sid * NC + cid
    base = wid * BW

    table_view = table_hbm.reshape(N_TILES, TILE_H, N_FACTORS)

    lane = lax.iota(jnp.int32, L)

    def phase(ph, _):
        pbase = base + ph * PH
        pltpu.sync_copy(items0_hbm.at[pl.ds(pbase, PH)], raw0_v)
        pltpu.sync_copy(items1_hbm.at[pl.ds(pbase, PH)], raw1_v)

        copies = []
        for g in range(NG):
            sl = pl.ds(g * L, L)
            t0v = lax.shift_right_logical(raw0_v[sl], 3)
            t1v = lax.shift_right_logical(raw1_v[sl], 3)
            for l in range(L):
                p = g * L + l
                copies.append(pltpu.async_copy(
                    table_view.at[t0v[l]], rows0_v.at[p], sem0))
                copies.append(pltpu.async_copy(
                    table_view.at[t1v[l]], rows1_v.at[p], sem1))
        for c in copies:
            c.wait()

        for g in range(NG):
            sl = pl.ds(g * L, L)
            p16 = jnp.full((L,), g * L, jnp.int32) + lane
            s0 = lax.bitwise_and(raw0_v[sl], 7)
            s1 = lax.bitwise_and(raw1_v[sl], 7)
            zero = jnp.zeros((L,), jnp.float32)

            def d_body(dd, acc):
                for j in range(8):
                    dvec = jnp.full((L,), dd * 8 + j, jnp.int32)
                    a = plsc.load_gather(rows0_v, [p16, s0, dvec])
                    b = plsc.load_gather(rows1_v, [p16, s1, dvec])
                    acc = acc + a * b
                return acc

            acc = lax.fori_loop(0, N_FACTORS // 8, d_body, zero,
                                unroll=False)
            out16 = 1.0 / (1.0 + jnp.exp(-acc))
            out_v[pl.ds(ph * PH + g * L, L)] = out16
        return 0

    lax.fori_loop(0, NPH, phase, 0, unroll=False)

    pltpu.sync_copy(out_v, out_hbm.at[pl.ds(base, BW)])


@jax.jit
def _newmf_sc(items0, items1, table):
    mesh = plsc.VectorSubcoreMesh(core_axis_name="c", subcore_axis_name="s")
    f = functools.partial(
        pl.kernel,
        out_type=jax.ShapeDtypeStruct((BATCH,), jnp.float32),
        mesh=mesh,
        scratch_types=[
            pltpu.VMEM((PH,), jnp.int32),
            pltpu.VMEM((PH,), jnp.int32),
            pltpu.VMEM((PH, TILE_H, N_FACTORS), jnp.float32),
            pltpu.VMEM((PH, TILE_H, N_FACTORS), jnp.float32),
            pltpu.VMEM((BW,), jnp.float32),
            pltpu.SemaphoreType.DMA,
            pltpu.SemaphoreType.DMA,
        ],
        compiler_params=pltpu.CompilerParams(
            use_tc_tiling_on_sc=True,
            needs_layout_passes=False,
        ),
    )(_body)
    return f(items0, items1, table)


def kernel(items, item_factors):
    items0 = items[0].astype(jnp.int32)
    items1 = items[1].astype(jnp.int32)
    return _newmf_sc(items0, items1, item_factors)
